# baseline (speedup 1.0000x reference)
import functools

import jax
import jax.numpy as jnp
from jax import lax
from jax.experimental import pallas as pl
from jax.experimental.pallas import tpu as pltpu
from jax.experimental.pallas import tpu_sc as plsc


def _lse_body(table_ref, lse_ref):
    t = table_ref[...]
    m = jnp.max(t, axis=1, keepdims=True)
    lse = (m + jnp.log(jnp.sum(jnp.exp(t - m), axis=1, keepdims=True)))[:, 0]
    lse_ref[...] = jnp.pad(lse, (0, lse_ref.shape[0] - lse.shape[0]))


def _compute_lse(table, vp):
    return pl.pallas_call(
        _lse_body,
        out_shape=jax.ShapeDtypeStruct((vp,), jnp.float32),
    )(table)


_INFO = plsc.get_sparse_core_info()
_NC, _NS, _L = _INFO.num_cores, _INFO.num_subcores, _INFO.num_lanes
_NW = _NC * _NS


@functools.lru_cache(maxsize=None)
def _make_sc_kernel(bsz, t, v, d):
    dp = ((d + 127) // 128) * 128
    vp = ((v + 127) // 128) * 128
    n = bsz * t
    ncb = (d + 7) // 8
    cb_per_w = (ncb + _NW - 1) // _NW
    tokw = n // _NW
    ngrp = tokw // _L
    nbb = bsz // _L
    assert t % 2 == 0 and bsz % _L == 0 and tokw % _L == 0
    mesh = plsc.VectorSubcoreMesh(core_axis_name="c", subcore_axis_name="s")

    @functools.partial(
        pl.kernel,
        mesh=mesh,
        compiler_params=pltpu.CompilerParams(
            needs_layout_passes=False, use_tc_tiling_on_sc=True
        ),
        out_type=(
            jax.ShapeDtypeStruct((t, d, bsz), jnp.float32),
            jax.ShapeDtypeStruct((_NW * _L,), jnp.float32),
        ),
        scratch_types=[
            pltpu.VMEM((8 * cb_per_w, vp), jnp.float32),
            pltpu.VMEM((8, bsz), jnp.float32),
            pltpu.VMEM((8, bsz), jnp.float32),
            pltpu.VMEM((bsz,), jnp.int32),
            pltpu.VMEM((bsz,), jnp.int32),
            pltpu.VMEM((tokw,), jnp.int32),
            pltpu.VMEM((tokw,), jnp.int32),
            pltpu.VMEM((dp,), jnp.float32),
            pltpu.VMEM((tokw,), jnp.int32),
            pltpu.VMEM((tokw,), jnp.float32),
            pltpu.VMEM((_L,), jnp.float32),
            pltpu.SemaphoreType.DMA,
            pltpu.SemaphoreType.DMA,
            pltpu.SemaphoreType.DMA,
            pltpu.SemaphoreType.DMA,
            pltpu.SemaphoreType.DMA,
        ],
    )
    def sc_kernel(idxt_hbm, idx_hbm, tgt_hbm, tabt_hbm, tabflat_hbm, lse_hbm,
                  out_hbm, part_hbm,
                  tabw, st0, st1, ib0, ib1, idxw, tgtw, lse_v, fi_v, vals_v,
                  acc_v, is0, is1, ws0, ws1, vsem):
        wid = lax.axis_index("s") * _NC + lax.axis_index("c")
        w_tok0 = wid * tokw
        cb0 = wid * cb_per_w
        stage = (st0, st1)
        ibuf = (ib0, ib1)
        isem = (is0, is1)
        wsem = (ws0, ws1)

        pltpu.sync_copy(lse_hbm, lse_v)
        pltpu.sync_copy(idx_hbm.at[pl.ds(w_tok0, tokw)], idxw)
        pltpu.sync_copy(tgt_hbm.at[pl.ds(w_tok0, tokw)], tgtw)

        def fi_body(g, carry):
            iv = idxw[pl.ds(g * _L, _L)]
            tv = tgtw[pl.ds(g * _L, _L)]
            fi_v[pl.ds(g * _L, _L)] = iv * dp + tv
            return carry

        lax.fori_loop(0, ngrp, fi_body, 0)
        pltpu.async_copy(tabflat_hbm.at[fi_v], vals_v, vsem)

        pltpu.sync_copy(tabt_hbm.at[pl.ds(cb0 * 8, 8 * cb_per_w)], tabw)

        def start_idx(tt, b):
            pltpu.async_copy(
                idxt_hbm.at[pl.ds(tt * bsz, bsz)], ibuf[b], isem[b])

        def wait_idx(b):
            pltpu.make_async_copy(
                idxt_hbm.at[pl.ds(0, bsz)], ibuf[b], isem[b]).wait()

        def wait_write(s):
            pltpu.make_async_copy(
                stage[s], out_hbm.at[0, pl.ds(0, 8), :], wsem[s]).wait()

        start_idx(0, 0)
        start_idx(1, 1)

        def do_t(tt, b):
            wait_idx(b)
            for cbl in range(cb_per_w):
                cb = cb0 + cbl
                s = cbl % 2

                @pl.when(cb < ncb)
                def _():
                    if cbl >= 2:
                        wait_write(s)
                    else:
                        @pl.when(tt > 0)
                        def _():
                            wait_write(s)

                    @plsc.parallel_loop(0, nbb, 1, unroll=8)
                    def _(bb):
                        iv = ibuf[b][pl.ds(bb * _L, _L)]
                        for ri in range(8):
                            rid = jnp.full((_L,), cbl * 8 + ri, jnp.int32)
                            val = plsc.load_gather(tabw, [rid, iv])
                            stage[s][ri, pl.ds(bb * _L, _L)] = val
                    pltpu.async_copy(
                        stage[s],
                        out_hbm.at[tt, pl.ds(cb * 8, 8), :],
                        wsem[s],
                    )

            @pl.when(tt + 2 < t)
            def _():
                start_idx(tt + 2, b)

        def body(g, carry):
            for b in range(2):
                do_t(g * 2 + b, b)
            return carry

        lax.fori_loop(0, t // 2, body, 0)
        for s in range(2):
            @pl.when(cb0 + s < ncb)
            def _():
                wait_write(s)

        pltpu.make_async_copy(
            tabflat_hbm.at[pl.ds(0, tokw)], vals_v, vsem).wait()

        def loss_body(g, acc):
            iv = idxw[pl.ds(g * _L, _L)]
            lse_g = plsc.load_gather(lse_v, [iv])
            vv = vals_v[pl.ds(g * _L, _L)]
            return acc + (lse_g - vv)

        acc = lax.fori_loop(0, ngrp, loss_body, jnp.zeros((_L,), jnp.float32))
        acc_v[...] = acc
        pltpu.sync_copy(acc_v, part_hbm.at[pl.ds(wid * _L, _L)])

    return sc_kernel


def kernel(idx, targets, table):
    bsz, t = idx.shape
    v, d = table.shape
    dp = ((d + 127) // 128) * 128
    bp = ((bsz + 127) // 128) * 128
    assert bp == bsz
    n = bsz * t
    idx_f = idx.reshape(n).astype(jnp.int32)
    tgt_f = targets.reshape(n).astype(jnp.int32)
    idxt_f = idx.T.astype(jnp.int32).reshape(t * bsz)
    ncb_pad = ((d + 7) // 8 + _NW - 1) // _NW * _NW * 8
    vpad = ((v + 127) // 128) * 128
    tabt = jnp.pad(table.T.astype(jnp.float32),
                   ((0, ncb_pad - d), (0, vpad - v)))
    tabflat = jnp.pad(table, ((0, 0), (0, dp - d))).reshape(v * dp)
    lse = _compute_lse(table, dp)
    out3, part = _make_sc_kernel(bsz, t, v, d)(
        idxt_f, idx_f, tgt_f, tabt, tabflat, lse)
    logits = out3.transpose(2, 0, 1)
    loss = jnp.sum(part) / n
    return (logits, loss)


# speedup vs baseline: 1.0551x; 1.0551x over previous
import functools

import jax
import jax.numpy as jnp
from jax import lax
from jax.experimental import pallas as pl
from jax.experimental.pallas import tpu as pltpu
from jax.experimental.pallas import tpu_sc as plsc


def _prep_body(table_ref, tabt_ref, lse_ref):
    t = table_ref[...]
    m = jnp.max(t, axis=1, keepdims=True)
    lse = (m + jnp.log(jnp.sum(jnp.exp(t - m), axis=1, keepdims=True)))[:, 0]
    lse_ref[...] = jnp.pad(lse, (0, lse_ref.shape[0] - lse.shape[0]))
    cp, vp = tabt_ref.shape
    v, d = t.shape
    tabt_ref[...] = jnp.pad(t, ((0, vp - v), (0, cp - d))).T


def _compute_prep(table, cp, vp):
    return pl.pallas_call(
        _prep_body,
        out_shape=(
            jax.ShapeDtypeStruct((cp, vp), jnp.float32),
            jax.ShapeDtypeStruct((vp,), jnp.float32),
        ),
    )(table)


_INFO = plsc.get_sparse_core_info()
_NC, _NS, _L = _INFO.num_cores, _INFO.num_subcores, _INFO.num_lanes
_NW = _NC * _NS


@functools.lru_cache(maxsize=None)
def _make_sc_kernel(bsz, t, v, d):
    dp = ((d + 127) // 128) * 128
    vp = ((v + 127) // 128) * 128
    n = bsz * t
    ncb = (d + 7) // 8
    cb_per_w = (ncb + _NW - 1) // _NW
    tokw = n // _NW
    ngrp = tokw // _L
    nbb = bsz // _L
    assert t % 2 == 0 and bsz % _L == 0 and tokw % _L == 0
    mesh = plsc.VectorSubcoreMesh(core_axis_name="c", subcore_axis_name="s")

    @functools.partial(
        pl.kernel,
        mesh=mesh,
        compiler_params=pltpu.CompilerParams(
            needs_layout_passes=False, use_tc_tiling_on_sc=True
        ),
        out_type=(
            jax.ShapeDtypeStruct((t, d, bsz), jnp.float32),
            jax.ShapeDtypeStruct((_NW * _L,), jnp.float32),
        ),
        scratch_types=[
            pltpu.VMEM((8 * cb_per_w, vp), jnp.float32),
            pltpu.VMEM((8, bsz), jnp.float32),
            pltpu.VMEM((8, bsz), jnp.float32),
            pltpu.VMEM((bsz,), jnp.int32),
            pltpu.VMEM((bsz,), jnp.int32),
            pltpu.VMEM((tokw,), jnp.int32),
            pltpu.VMEM((tokw,), jnp.int32),
            pltpu.VMEM((dp,), jnp.float32),
            pltpu.VMEM((tokw,), jnp.int32),
            pltpu.VMEM((tokw,), jnp.float32),
            pltpu.VMEM((_L,), jnp.float32),
            pltpu.SemaphoreType.DMA,
            pltpu.SemaphoreType.DMA,
            pltpu.SemaphoreType.DMA,
            pltpu.SemaphoreType.DMA,
            pltpu.SemaphoreType.DMA,
        ],
    )
    def sc_kernel(idxt_hbm, idx_hbm, tgt_hbm, tabt_hbm, tabflat_hbm, lse_hbm,
                  out_hbm, part_hbm,
                  tabw, st0, st1, ib0, ib1, idxw, tgtw, lse_v, fi_v, vals_v,
                  acc_v, is0, is1, ws0, ws1, vsem):
        wid = lax.axis_index("s") * _NC + lax.axis_index("c")
        w_tok0 = wid * tokw
        cb0 = wid * cb_per_w
        stage = (st0, st1)
        ibuf = (ib0, ib1)
        isem = (is0, is1)
        wsem = (ws0, ws1)

        pltpu.sync_copy(lse_hbm, lse_v)
        pltpu.sync_copy(idx_hbm.at[pl.ds(w_tok0, tokw)], idxw)
        pltpu.sync_copy(tgt_hbm.at[pl.ds(w_tok0, tokw)], tgtw)

        def fi_body(g, carry):
            iv = idxw[pl.ds(g * _L, _L)]
            tv = tgtw[pl.ds(g * _L, _L)]
            fi_v[pl.ds(g * _L, _L)] = iv * dp + tv
            return carry

        lax.fori_loop(0, ngrp, fi_body, 0)
        pltpu.async_copy(tabflat_hbm.at[fi_v], vals_v, vsem)

        pltpu.sync_copy(tabt_hbm.at[pl.ds(cb0 * 8, 8 * cb_per_w)], tabw)

        def start_idx(tt, b):
            pltpu.async_copy(
                idxt_hbm.at[pl.ds(tt * bsz, bsz)], ibuf[b], isem[b])

        def wait_idx(b):
            pltpu.make_async_copy(
                idxt_hbm.at[pl.ds(0, bsz)], ibuf[b], isem[b]).wait()

        def wait_write(s):
            pltpu.make_async_copy(
                stage[s], out_hbm.at[0, pl.ds(0, 8), :], wsem[s]).wait()

        start_idx(0, 0)
        start_idx(1, 1)

        def do_t(tt, b):
            wait_idx(b)
            for cbl in range(cb_per_w):
                cb = cb0 + cbl
                s = cbl % 2

                @pl.when(cb < ncb)
                def _():
                    if cbl >= 2:
                        wait_write(s)
                    else:
                        @pl.when(tt > 0)
                        def _():
                            wait_write(s)

                    @plsc.parallel_loop(0, nbb, 1, unroll=4)
                    def _(bb):
                        iv = ibuf[b][pl.ds(bb * _L, _L)]
                        for ri in range(8):
                            rid = jnp.full((_L,), cbl * 8 + ri, jnp.int32)
                            val = plsc.load_gather(tabw, [rid, iv])
                            stage[s][ri, pl.ds(bb * _L, _L)] = val
                    pltpu.async_copy(
                        stage[s],
                        out_hbm.at[tt, pl.ds(cb * 8, 8), :],
                        wsem[s],
                    )

            @pl.when(tt + 2 < t)
            def _():
                start_idx(tt + 2, b)

        def body(g, carry):
            for b in range(2):
                do_t(g * 2 + b, b)
            return carry

        lax.fori_loop(0, t // 2, body, 0)
        for s in range(2):
            @pl.when(cb0 + s < ncb)
            def _():
                wait_write(s)

        pltpu.make_async_copy(
            tabflat_hbm.at[pl.ds(0, tokw)], vals_v, vsem).wait()

        def loss_body(g, acc):
            iv = idxw[pl.ds(g * _L, _L)]
            lse_g = plsc.load_gather(lse_v, [iv])
            vv = vals_v[pl.ds(g * _L, _L)]
            return acc + (lse_g - vv)

        acc = lax.fori_loop(0, ngrp, loss_body, jnp.zeros((_L,), jnp.float32))
        acc_v[...] = acc
        pltpu.sync_copy(acc_v, part_hbm.at[pl.ds(wid * _L, _L)])

    return sc_kernel


def kernel(idx, targets, table):
    bsz, t = idx.shape
    v, d = table.shape
    dp = ((d + 127) // 128) * 128
    bp = ((bsz + 127) // 128) * 128
    assert bp == bsz
    n = bsz * t
    idx_f = idx.reshape(n).astype(jnp.int32)
    tgt_f = targets.reshape(n).astype(jnp.int32)
    idxt_f = idx.T.astype(jnp.int32).reshape(t * bsz)
    ncb_pad = ((d + 7) // 8 + _NW - 1) // _NW * _NW * 8
    vpad = ((v + 127) // 128) * 128
    tabflat = jnp.pad(table, ((0, 0), (0, dp - d))).reshape(v * dp)
    tabt, lse = _compute_prep(table.astype(jnp.float32), ncb_pad, vpad)
    out3, part = _make_sc_kernel(bsz, t, v, d)(
        idxt_f, idx_f, tgt_f, tabt, tabflat, lse)
    logits = out3.transpose(2, 0, 1)
    loss = jnp.sum(part) / n
    return (logits, loss)


# speedup vs baseline: 1.0707x; 1.0148x over previous
import functools

import jax
import jax.numpy as jnp
from jax import lax
from jax.experimental import pallas as pl
from jax.experimental.pallas import tpu as pltpu
from jax.experimental.pallas import tpu_sc as plsc


def _prep_body(table_ref, tabt_ref, lse_ref):
    t = table_ref[...]
    m = jnp.max(t, axis=1, keepdims=True)
    lse = (m + jnp.log(jnp.sum(jnp.exp(t - m), axis=1, keepdims=True)))[:, 0]
    lse_ref[...] = jnp.pad(lse, (0, lse_ref.shape[0] - lse.shape[0]))
    cp, vp = tabt_ref.shape
    v, d = t.shape
    tabt_ref[...] = jnp.pad(t, ((0, vp - v), (0, cp - d))).T


def _compute_prep(table, cp, vp):
    return pl.pallas_call(
        _prep_body,
        out_shape=(
            jax.ShapeDtypeStruct((cp, vp), jnp.float32),
            jax.ShapeDtypeStruct((vp,), jnp.float32),
        ),
    )(table)


_INFO = plsc.get_sparse_core_info()
_NC, _NS, _L = _INFO.num_cores, _INFO.num_subcores, _INFO.num_lanes
_NW = _NC * _NS


@functools.lru_cache(maxsize=None)
def _make_sc_kernel(bsz, t, v, d):
    dp = ((d + 127) // 128) * 128
    vp = ((v + 127) // 128) * 128
    n = bsz * t
    ncb = (d + 7) // 8
    cb_per_w = (ncb + _NW - 1) // _NW
    tokw = n // _NW
    ngrp = tokw // _L
    nbb = bsz // _L
    assert t % 2 == 0 and bsz % _L == 0 and tokw % _L == 0
    mesh = plsc.VectorSubcoreMesh(core_axis_name="c", subcore_axis_name="s")

    @functools.partial(
        pl.kernel,
        mesh=mesh,
        compiler_params=pltpu.CompilerParams(
            needs_layout_passes=False, use_tc_tiling_on_sc=True
        ),
        out_type=(
            jax.ShapeDtypeStruct((t, d, bsz), jnp.float32),
            jax.ShapeDtypeStruct((_NW * _L,), jnp.float32),
        ),
        scratch_types=[
            pltpu.VMEM((8 * cb_per_w, vp), jnp.float32),
            pltpu.VMEM((8, bsz), jnp.float32),
            pltpu.VMEM((8, bsz), jnp.float32),
            pltpu.VMEM((bsz,), jnp.int32),
            pltpu.VMEM((bsz,), jnp.int32),
            pltpu.VMEM((tokw,), jnp.int32),
            pltpu.VMEM((tokw,), jnp.int32),
            pltpu.VMEM((dp,), jnp.float32),
            pltpu.VMEM((tokw,), jnp.int32),
            pltpu.VMEM((tokw,), jnp.float32),
            pltpu.VMEM((_L,), jnp.float32),
            pltpu.SemaphoreType.DMA,
            pltpu.SemaphoreType.DMA,
            pltpu.SemaphoreType.DMA,
            pltpu.SemaphoreType.DMA,
            pltpu.SemaphoreType.DMA,
        ],
    )
    def sc_kernel(idxt_hbm, tgt_hbm, tabt_hbm, tabflat_hbm, lse_hbm,
                  out_hbm, part_hbm,
                  tabw, st0, st1, ib0, ib1, idxw, tgtw, lse_v, fi_v, vals_v,
                  acc_v, is0, is1, ws0, ws1, vsem):
        wid = lax.axis_index("s") * _NC + lax.axis_index("c")
        w_tok0 = wid * tokw
        cb0 = wid * cb_per_w
        stage = (st0, st1)
        ibuf = (ib0, ib1)
        isem = (is0, is1)
        wsem = (ws0, ws1)

        pltpu.sync_copy(lse_hbm, lse_v)
        pltpu.sync_copy(idxt_hbm.at[pl.ds(w_tok0, tokw)], idxw)
        pltpu.sync_copy(tgt_hbm.at[pl.ds(w_tok0, tokw)], tgtw)

        def fi_body(g, carry):
            iv = idxw[pl.ds(g * _L, _L)]
            tv = tgtw[pl.ds(g * _L, _L)]
            fi_v[pl.ds(g * _L, _L)] = iv * dp + tv
            return carry

        lax.fori_loop(0, ngrp, fi_body, 0)
        pltpu.async_copy(tabflat_hbm.at[fi_v], vals_v, vsem)

        pltpu.sync_copy(tabt_hbm.at[pl.ds(cb0 * 8, 8 * cb_per_w)], tabw)

        def start_idx(tt, b):
            pltpu.async_copy(
                idxt_hbm.at[pl.ds(tt * bsz, bsz)], ibuf[b], isem[b])

        def wait_idx(b):
            pltpu.make_async_copy(
                idxt_hbm.at[pl.ds(0, bsz)], ibuf[b], isem[b]).wait()

        def wait_write(s):
            pltpu.make_async_copy(
                stage[s], out_hbm.at[0, pl.ds(0, 8), :], wsem[s]).wait()

        start_idx(0, 0)
        start_idx(1, 1)

        def do_t(tt, b):
            wait_idx(b)
            for cbl in range(cb_per_w):
                cb = cb0 + cbl
                s = cbl % 2

                @pl.when(cb < ncb)
                def _():
                    if cbl >= 2:
                        wait_write(s)
                    else:
                        @pl.when(tt > 0)
                        def _():
                            wait_write(s)

                    @plsc.parallel_loop(0, nbb, 1, unroll=4)
                    def _(bb):
                        iv = ibuf[b][pl.ds(bb * _L, _L)]
                        for ri in range(8):
                            rid = jnp.full((_L,), cbl * 8 + ri, jnp.int32)
                            val = plsc.load_gather(tabw, [rid, iv])
                            stage[s][ri, pl.ds(bb * _L, _L)] = val
                    pltpu.async_copy(
                        stage[s],
                        out_hbm.at[tt, pl.ds(cb * 8, 8), :],
                        wsem[s],
                    )

            @pl.when(tt + 2 < t)
            def _():
                start_idx(tt + 2, b)

        def body(g, carry):
            for b in range(2):
                do_t(g * 2 + b, b)
            return carry

        lax.fori_loop(0, t // 2, body, 0)
        for s in range(2):
            @pl.when(cb0 + s < ncb)
            def _():
                wait_write(s)

        pltpu.make_async_copy(
            tabflat_hbm.at[pl.ds(0, tokw)], vals_v, vsem).wait()

        def loss_body(g, acc):
            iv = idxw[pl.ds(g * _L, _L)]
            lse_g = plsc.load_gather(lse_v, [iv])
            vv = vals_v[pl.ds(g * _L, _L)]
            return acc + (lse_g - vv)

        acc = lax.fori_loop(0, ngrp, loss_body, jnp.zeros((_L,), jnp.float32))
        acc_v[...] = acc
        pltpu.sync_copy(acc_v, part_hbm.at[pl.ds(wid * _L, _L)])

    return sc_kernel


def kernel(idx, targets, table):
    bsz, t = idx.shape
    v, d = table.shape
    dp = ((d + 127) // 128) * 128
    bp = ((bsz + 127) // 128) * 128
    assert bp == bsz
    n = bsz * t
    idxt_f = idx.T.astype(jnp.int32).reshape(t * bsz)
    tgtt_f = targets.T.astype(jnp.int32).reshape(t * bsz)
    ncb_pad = ((d + 7) // 8 + _NW - 1) // _NW * _NW * 8
    vpad = ((v + 127) // 128) * 128
    tabflat = jnp.pad(table, ((0, 0), (0, dp - d))).reshape(v * dp)
    tabt, lse = _compute_prep(table.astype(jnp.float32), ncb_pad, vpad)
    out3, part = _make_sc_kernel(bsz, t, v, d)(
        idxt_f, tgtt_f, tabt, tabflat, lse)
    logits = out3.transpose(2, 0, 1)
    loss = jnp.sum(part) / n
    return (logits, loss)
